# baseline (speedup 1.0000x reference)
import functools

import jax
import jax.numpy as jnp
from jax.experimental import pallas as pl

N = 10000
E = 100000
D = 768
H = 8
DH = D // H

_MBLK = 1000


def _mm2_body(x_ref, ws_ref, wd_ref, hs_ref, hd_ref):
    x = x_ref[...]
    hs_ref[...] = jnp.dot(x, ws_ref[...], preferred_element_type=jnp.float32)
    hd_ref[...] = jnp.dot(x, wd_ref[...], preferred_element_type=jnp.float32)


@jax.jit
def _mm2(x, ws, wd):
    grid = (N // _MBLK,)
    return pl.pallas_call(
        _mm2_body,
        grid=grid,
        in_specs=[
            pl.BlockSpec((_MBLK, D), lambda i: (i, 0)),
            pl.BlockSpec((D, D), lambda i: (0, 0)),
            pl.BlockSpec((D, D), lambda i: (0, 0)),
        ],
        out_specs=[
            pl.BlockSpec((_MBLK, D), lambda i: (i, 0)),
            pl.BlockSpec((_MBLK, D), lambda i: (i, 0)),
        ],
        out_shape=[
            jax.ShapeDtypeStruct((N, D), jnp.float32),
            jax.ShapeDtypeStruct((N, D), jnp.float32),
        ],
    )(x, ws, wd)


def _head_body(x0_ref, w1_ref, b1_ref, w2_ref, b2_ref, h_ref, y_ref):
    h = jnp.dot(x0_ref[...], w1_ref[...], preferred_element_type=jnp.float32)
    h = h + b1_ref[...]
    h = jnp.where(h >= 0, h, 0.01 * h)
    h_ref[...] = h
    y_ref[...] = jnp.dot(h, w2_ref[...], preferred_element_type=jnp.float32) + b2_ref[...]


@jax.jit
def _head(x0, w1, b1, w2, b2):
    return pl.pallas_call(
        _head_body,
        out_shape=[
            jax.ShapeDtypeStruct((8, 600), jnp.float32),
            jax.ShapeDtypeStruct((8, 1), jnp.float32),
        ],
    )(x0, w1, b1.reshape(1, 600), w2, b2.reshape(1, 1))


def _edge_phase(hs, hd, src, dst, attn, bias):
    hs3 = hs.reshape(N, H, DH)
    hd3 = hd.reshape(N, H, DH)
    e = jax.nn.leaky_relu(hs3[src] + hd3[dst], negative_slope=0.2)
    logits = jnp.sum(e * attn[None, :, :], axis=-1)
    m = jax.ops.segment_max(logits, dst, num_segments=N)
    m = jnp.where(jnp.isfinite(m), m, 0.0)
    ex = jnp.exp(logits - m[dst])
    den = jax.ops.segment_sum(ex, dst, num_segments=N)
    alpha = ex / jnp.maximum(den[dst], 1e-9)
    out = jax.ops.segment_sum(hs3[src] * alpha[:, :, None], dst, num_segments=N)
    return (out + bias[None, :, :]).reshape(N, D)


def kernel(features, edge_index, Wsrc0, Wdst0, attn0, bias0, Wsrc1, Wdst1, attn1, bias1, Wsrc2, Wdst2, attn2, bias2, Wsrc3, Wdst3, attn3, bias3, W1, b1, W2, b2):
    loop = jnp.arange(N, dtype=edge_index.dtype)
    src = jnp.concatenate([edge_index[0], loop])
    dst = jnp.concatenate([edge_index[1], loop])
    x = features
    for (Ws, Wd, a, b) in ((Wsrc0, Wdst0, attn0, bias0), (Wsrc1, Wdst1, attn1, bias1), (Wsrc2, Wdst2, attn2, bias2), (Wsrc3, Wdst3, attn3, bias3)):
        hs, hd = _mm2(x, Ws, Wd)
        x = _edge_phase(hs, hd, src, dst, a, b)
    h8, y8 = _head(x[0:8], W1, b1, W2, b2)
    return (h8[0], y8[0])
